# baseline (speedup 1.0000x reference)
import jax
import jax.numpy as jnp
from jax.experimental import pallas as pl
from jax.experimental.pallas import tpu as pltpu


def _round_up(x: int, m: int) -> int:
    return ((x + m - 1) // m) * m


def _fc_nt_kernel(x_ref, w_ref, b_ref, out_ref):
    x = x_ref[...]
    w = w_ref[...].astype(jnp.bfloat16)
    acc = jax.lax.dot_general(
        x, w, dimension_numbers=(((1,), (1,)), ((), ())),
        preferred_element_type=jnp.float32)
    out_ref[...] = (acc + b_ref[...]).astype(out_ref.dtype)


@jax.jit
def kernel(x_nchw, weight, bias):
    n = x_nchw.shape[0]
    x_flat = jnp.reshape(x_nchw, (n, -1))
    num_ftrs = x_flat.shape[1]
    num_classes = weight.shape[0]
    out_dtype = x_flat.dtype

    b2d = bias.astype(jnp.float32).reshape(1, num_classes)

    tm = min(1024, _round_up(max(n, 8), 8))
    n_pad = _round_up(n, tm)
    x_bf = x_flat.astype(jnp.bfloat16)
    x_p = x_bf if n_pad == n else jnp.pad(x_bf, ((0, n_pad - n), (0, 0)))

    out3_p = pl.pallas_call(
        _fc_nt_kernel,
        out_shape=jax.ShapeDtypeStruct((n_pad, num_classes), out_dtype),
        grid=(n_pad // tm,),
        in_specs=[
            pl.BlockSpec((tm, num_ftrs), lambda i: (i, 0)),
            pl.BlockSpec((num_classes, num_ftrs), lambda i: (0, 0)),
            pl.BlockSpec((1, num_classes), lambda i: (0, 0)),
        ],
        out_specs=pl.BlockSpec((tm, num_classes), lambda i: (i, 0)),
        compiler_params=pltpu.CompilerParams(
            dimension_semantics=("parallel",),
            vmem_limit_bytes=48 * 1024 * 1024,
        ),
    )(x_p, weight, b2d)

    out1 = jnp.copy(x_flat)
    if n_pad == n:
        return out1, out3_p
    return out1, out3_p[:n]


# speedup vs baseline: 1.3892x; 1.3892x over previous
import jax
import jax.numpy as jnp
from jax.experimental import pallas as pl
from jax.experimental.pallas import tpu as pltpu


def _round_up(x: int, m: int) -> int:
    return ((x + m - 1) // m) * m


def _fc_nt_kernel(x_ref, w_ref, b_ref, out_ref):
    x = x_ref[...].astype(jnp.bfloat16)
    w = w_ref[...].astype(jnp.bfloat16)
    acc = jax.lax.dot_general(
        x, w, dimension_numbers=(((1,), (1,)), ((), ())),
        preferred_element_type=jnp.float32)
    out_ref[...] = (acc + b_ref[...]).astype(out_ref.dtype)


@jax.jit
def kernel(x_nchw, weight, bias):
    n = x_nchw.shape[0]
    x_flat = jnp.reshape(x_nchw, (n, -1))
    num_ftrs = x_flat.shape[1]
    num_classes = weight.shape[0]
    out_dtype = x_flat.dtype

    b2d = bias.astype(jnp.float32).reshape(1, num_classes)

    tm = min(1024, _round_up(max(n, 8), 8))
    n_pad = _round_up(n, tm)
    x_p = x_flat if n_pad == n else jnp.pad(x_flat, ((0, n_pad - n), (0, 0)))

    out3_p = pl.pallas_call(
        _fc_nt_kernel,
        out_shape=jax.ShapeDtypeStruct((n_pad, num_classes), out_dtype),
        grid=(n_pad // tm,),
        in_specs=[
            pl.BlockSpec((tm, num_ftrs), lambda i: (i, 0)),
            pl.BlockSpec((num_classes, num_ftrs), lambda i: (0, 0)),
            pl.BlockSpec((1, num_classes), lambda i: (0, 0)),
        ],
        out_specs=pl.BlockSpec((tm, num_classes), lambda i: (i, 0)),
        compiler_params=pltpu.CompilerParams(
            dimension_semantics=("parallel",),
            vmem_limit_bytes=48 * 1024 * 1024,
        ),
    )(x_p, weight, b2d)

    out1 = jnp.copy(x_flat)
    if n_pad == n:
        return out1, out3_p
    return out1, out3_p[:n]
